# baseline (speedup 1.0000x reference)
import functools

import jax
import jax.numpy as jnp
from jax import lax
from jax.experimental import pallas as pl
from jax.experimental.pallas import tpu as pltpu
from jax.experimental.pallas import tpu_sc as plsc

NUM_EMB = 1000000
EMB_DIM = 64
RANK = 16
SCALING = 32.0 / 16.0
NUM_CORES = 2
NUM_SUBCORES = 16
NW = NUM_CORES * NUM_SUBCORES
CHUNK = 128
PACK_BN = 8192
PACK_C = 499712


def _fuse_block(wtA_ref, wtB_ref, atA_ref, atB_ref, bt_ref, f_ref):
  cA = wtA_ref[...] + SCALING * jnp.dot(
      bt_ref[...], atA_ref[...], preferred_element_type=jnp.float32)
  cB = wtB_ref[...] + SCALING * jnp.dot(
      bt_ref[...], atB_ref[...], preferred_element_type=jnp.float32)
  f_ref[...] = jnp.concatenate([cA.T, cB.T], axis=1)


def _build_fused_table(weight, lora_A, lora_B):
  n = weight.shape[0]
  nb = PACK_C // PACK_BN
  n2 = n - PACK_C
  grid = (n2 + PACK_BN - 1) // PACK_BN
  return pl.pallas_call(
      _fuse_block,
      grid=(grid,),
      in_specs=[
          pl.BlockSpec((EMB_DIM, PACK_BN), lambda i: (0, i)),
          pl.BlockSpec((EMB_DIM, PACK_BN), lambda i: (0, i + nb)),
          pl.BlockSpec((RANK, PACK_BN), lambda i: (0, i)),
          pl.BlockSpec((RANK, PACK_BN), lambda i: (0, i + nb)),
          pl.BlockSpec((EMB_DIM, RANK), lambda i: (0, 0)),
      ],
      out_specs=pl.BlockSpec((PACK_BN, 2 * EMB_DIM), lambda i: (i, 0)),
      out_shape=jax.ShapeDtypeStruct((n2, 2 * EMB_DIM), jnp.float32),
  )(weight.T, weight.T, lora_A.T, lora_A.T, lora_B.T)


def _make_gather(n_idx):
  assert n_idx % (NW * 4 * CHUNK) == 0
  per_w = n_idx // NW
  n_step = per_w // (2 * CHUNK)
  mesh = plsc.VectorSubcoreMesh(core_axis_name="c", subcore_axis_name="s")

  @functools.partial(
      pl.kernel,
      out_type=jax.ShapeDtypeStruct((n_idx, EMB_DIM), jnp.float32),
      mesh=mesh,
      scratch_types=[
          pltpu.VMEM((n_step, 2, CHUNK), jnp.int32),
          pltpu.VMEM((2, 2 * CHUNK, EMB_DIM), jnp.float32),
          pltpu.SemaphoreType.DMA,
          pltpu.SemaphoreType.DMA,
          pltpu.SemaphoreType.DMA,
          pltpu.SemaphoreType.DMA,
      ],
      compiler_params=pltpu.CompilerParams(use_tc_tiling_on_sc=False),
  )
  def gather_rows(x_hbm, f_hbm, out_hbm, idx_all, rows_v, gs0, gs1, ws0, ws1):
    gs = (gs0, gs1)
    ws = (ws0, ws1)
    wid = lax.axis_index("s") * NUM_CORES + lax.axis_index("c")
    base = wid * per_w
    pltpu.sync_copy(x_hbm.at[wid], idx_all)

    def issue(i, s):
      pltpu.async_copy(
          f_hbm.at[idx_all.at[i, 0]], rows_v.at[s, pl.ds(0, CHUNK)], gs[s])
      pltpu.async_copy(
          f_hbm.at[idx_all.at[i, 1]], rows_v.at[s, pl.ds(CHUNK, CHUNK)],
          gs[s])

    def drain_gather(s):
      pltpu.make_async_copy(
          f_hbm.at[pl.ds(0, 2 * CHUNK)], rows_v.at[s], gs[s]).wait()

    def drain_write(s):
      pltpu.make_async_copy(
          rows_v.at[s], out_hbm.at[pl.ds(0, 2 * CHUNK)], ws[s]).wait()

    issue(0, 0)

    def outer(g, carry):
      for b in range(2):
        i = 2 * g + b
        nxt = i + 1

        @pl.when(nxt < n_step)
        def _():
          if b == 1:
            drain_write(0)
          else:

            @pl.when(i >= 1)
            def _():
              drain_write(1)

          issue(nxt, 1 - b)

        drain_gather(b)
        off = base + i * 2 * CHUNK
        pltpu.async_copy(rows_v.at[b], out_hbm.at[pl.ds(off, 2 * CHUNK)],
                         ws[b])
      return carry

    lax.fori_loop(0, n_step // 2, outer, 0, unroll=False)
    drain_write(0)
    drain_write(1)

  return gather_rows


FIN_BN = 400


def _finalize_block(g_ref, o_ref):
  h = pl.program_id(1)
  c = g_ref[...]
  sel = jnp.where(h == 0, c[:, :EMB_DIM], c[:, EMB_DIM:])
  o_ref[...] = sel.reshape(o_ref.shape)


def _finalize(gathered, n_b, n_l):
  half = gathered.shape[0]
  bm = FIN_BN // n_l
  return pl.pallas_call(
      _finalize_block,
      grid=(half // FIN_BN, 2),
      in_specs=[pl.BlockSpec((FIN_BN, 2 * EMB_DIM), lambda i, h: (i, 0))],
      out_specs=pl.BlockSpec(
          (bm, n_l, EMB_DIM), lambda i, h: ((half // FIN_BN) * h + i, 0, 0)),
      out_shape=jax.ShapeDtypeStruct((n_b, n_l, EMB_DIM), jnp.float32),
  )(gathered)


def kernel(x, weight, lora_A, lora_B):
  b, l = x.shape
  n = b * l
  half = n // 2
  xi = x.astype(jnp.int32)
  xj = jnp.where(xi < PACK_C, 2 * xi, 2 * (xi - PACK_C) + 1).reshape(n)
  xs = jnp.stack([xj[:half], xj[half:]], axis=1).reshape(n)
  xf = xs.reshape(NW, n // (NW * 2 * CHUNK), 2, CHUNK)
  fused2 = _build_fused_table(weight, lora_A, lora_B)
  fused = fused2.reshape(2 * fused2.shape[0], EMB_DIM)
  gathered = _make_gather(n)(xf, fused)
  return _finalize(gathered.reshape(half, 2 * EMB_DIM), b, l)


# speedup vs baseline: 2.3149x; 2.3149x over previous
import functools

import jax
import jax.numpy as jnp
from jax import lax
from jax.experimental import pallas as pl
from jax.experimental.pallas import tpu as pltpu
from jax.experimental.pallas import tpu_sc as plsc

NUM_EMB = 1000000
EMB_DIM = 64
RANK = 16
SCALING = 32.0 / 16.0
NUM_CORES = 2
NUM_SUBCORES = 16
NW = NUM_CORES * NUM_SUBCORES
CHUNK = 128
PACK_BN = 8192
PACK_C = 499712


def _fuse_block(wtA_ref, wtB_ref, atA_ref, atB_ref, bt_ref, f_ref):
  cA = wtA_ref[...] + SCALING * jnp.dot(
      bt_ref[...], atA_ref[...], preferred_element_type=jnp.float32)
  cB = wtB_ref[...] + SCALING * jnp.dot(
      bt_ref[...], atB_ref[...], preferred_element_type=jnp.float32)
  f_ref[...] = jnp.concatenate([cA.T, cB.T], axis=1)


def _build_fused_table(weight, lora_A, lora_B):
  n = weight.shape[0]
  nb = PACK_C // PACK_BN
  n2 = n - PACK_C
  grid = (n2 + PACK_BN - 1) // PACK_BN
  return pl.pallas_call(
      _fuse_block,
      grid=(grid,),
      in_specs=[
          pl.BlockSpec((EMB_DIM, PACK_BN), lambda i: (0, i)),
          pl.BlockSpec((EMB_DIM, PACK_BN), lambda i: (0, i + nb)),
          pl.BlockSpec((RANK, PACK_BN), lambda i: (0, i)),
          pl.BlockSpec((RANK, PACK_BN), lambda i: (0, i + nb)),
          pl.BlockSpec((EMB_DIM, RANK), lambda i: (0, 0)),
      ],
      out_specs=pl.BlockSpec((PACK_BN, 2 * EMB_DIM), lambda i: (i, 0)),
      out_shape=jax.ShapeDtypeStruct((n2, 2 * EMB_DIM), jnp.float32),
  )(weight.T, weight.T, lora_A.T, lora_A.T, lora_B.T)


def _make_gather(n_idx):
  assert n_idx % (NW * 2 * CHUNK) == 0
  per_w = n_idx // NW
  n_step = per_w // CHUNK
  mesh = plsc.VectorSubcoreMesh(core_axis_name="c", subcore_axis_name="s")

  @functools.partial(
      pl.kernel,
      out_type=jax.ShapeDtypeStruct((n_idx, EMB_DIM), jnp.float32),
      mesh=mesh,
      scratch_types=[
          pltpu.VMEM((n_step, CHUNK), jnp.int32),
          pltpu.VMEM((2, CHUNK, EMB_DIM), jnp.float32),
          pltpu.SemaphoreType.DMA,
          pltpu.SemaphoreType.DMA,
          pltpu.SemaphoreType.DMA,
          pltpu.SemaphoreType.DMA,
      ],
      compiler_params=pltpu.CompilerParams(use_tc_tiling_on_sc=False),
  )
  def gather_rows(x_hbm, f_hbm, out_hbm, idx_all, rows_v, gs0, gs1, ws0, ws1):
    gs = (gs0, gs1)
    ws = (ws0, ws1)
    wid = lax.axis_index("s") * NUM_CORES + lax.axis_index("c")
    base = wid * per_w
    pltpu.sync_copy(x_hbm.at[wid], idx_all)

    def issue(i, s):
      pltpu.async_copy(f_hbm.at[idx_all.at[i]], rows_v.at[s], gs[s])

    def drain_gather(s):
      pltpu.make_async_copy(f_hbm.at[pl.ds(0, CHUNK)], rows_v.at[s], gs[s]).wait()

    def drain_write(s):
      pltpu.make_async_copy(
          rows_v.at[s], out_hbm.at[pl.ds(0, CHUNK)], ws[s]).wait()

    issue(0, 0)

    def outer(g, carry):
      for b in range(2):
        i = 2 * g + b
        nxt = i + 1

        @pl.when(nxt < n_step)
        def _():
          if b == 1:
            drain_write(0)
          else:

            @pl.when(i >= 1)
            def _():
              drain_write(1)

          issue(nxt, 1 - b)

        drain_gather(b)
        off = base + i * CHUNK
        pltpu.async_copy(rows_v.at[b], out_hbm.at[pl.ds(off, CHUNK)], ws[b])
      return carry

    lax.fori_loop(0, n_step // 2, outer, 0, unroll=False)
    drain_write(0)
    drain_write(1)

  return gather_rows


def kernel(x, weight, lora_A, lora_B):
  b, l = x.shape
  n = b * l
  xi = x.astype(jnp.int32)
  xj = jnp.where(xi < PACK_C, 2 * xi, 2 * (xi - PACK_C) + 1)
  xf = xj.reshape(NW, n // (NW * CHUNK), CHUNK)
  fused2 = _build_fused_table(weight, lora_A, lora_B)
  fused = fused2.reshape(2 * fused2.shape[0], EMB_DIM)
  out = _make_gather(n)(xf, fused)
  return out.reshape(b, l, EMB_DIM)
